# baseline (speedup 1.0000x reference)
import functools

import jax
import jax.numpy as jnp
from jax import lax
from jax.experimental import pallas as pl
from jax.experimental.pallas import tpu as pltpu
from jax.experimental.pallas import tpu_sc as plsc

HASH_SIZE = 1000000
DIM = 32
OUT_DIM = 128
BATCH = 4
SEQ = 4096
TOT = BATCH * SEQ
NW = 32
CHUNK = TOT // NW
LANES = 16
IDX_ROWS = CHUNK // 128

_mesh = plsc.VectorSubcoreMesh(core_axis_name="c", subcore_axis_name="s")


@functools.partial(
    pl.kernel,
    mesh=_mesh,
    out_type=jax.ShapeDtypeStruct((TOT, DIM), jnp.float32),
    compiler_params=pltpu.CompilerParams(use_tc_tiling_on_sc=False),
    scratch_types=[
        pltpu.VMEM((CHUNK + 8,), jnp.int32),
        pltpu.VMEM((IDX_ROWS, 128), jnp.int32),
        pltpu.VMEM((CHUNK, DIM), jnp.float32),
        pltpu.SemaphoreType.DMA,
    ],
)
def _sc_hash_gather(tok_hbm, table_hbm, out_hbm, ext_v, idx_v, rows_v, sem):
    wid = lax.axis_index("s") * 2 + lax.axis_index("c")
    base = wid * CHUNK
    col = base % SEQ

    @pl.when(wid == 0)
    def _():
        pltpu.sync_copy(tok_hbm.at[pl.ds(0, CHUNK)], ext_v.at[pl.ds(8, CHUNK)])

    @pl.when(wid != 0)
    def _():
        pltpu.sync_copy(tok_hbm.at[pl.ds(base - 8, CHUNK + 8)], ext_v)

    lane = lax.iota(jnp.int32, 16)
    nz = jnp.minimum(col, 1)
    for q in range(CHUNK // LANES):
        prev = ext_v[pl.ds(7 + q * LANES, LANES)]
        cur = ext_v[pl.ds(8 + q * LANES, LANES)]
        h = (prev * 31 + cur) % HASH_SIZE
        if q == 0:
            h = h * jnp.minimum(lane + nz, 1)
        idx_v[q // 8, pl.ds((q % 8) * LANES, LANES)] = h

    copies = [
        pltpu.async_copy(
            table_hbm.at[idx_v.at[j]], rows_v.at[pl.ds(j * 128, 128)], sem
        )
        for j in range(IDX_ROWS)
    ]
    for cp in copies:
        cp.wait()

    pltpu.sync_copy(rows_v, out_hbm.at[pl.ds(base, CHUNK)])


def _mm_body(e_ref, wt_ref, o_ref):
    o_ref[...] = jnp.dot(
        e_ref[...], wt_ref[...], preferred_element_type=jnp.float32
    )


_MM_BLK = 2048
_mm = pl.pallas_call(
    _mm_body,
    grid=(TOT // _MM_BLK,),
    in_specs=[
        pl.BlockSpec((_MM_BLK, DIM), lambda i: (i, 0)),
        pl.BlockSpec((DIM, OUT_DIM), lambda i: (0, 0)),
    ],
    out_specs=pl.BlockSpec((_MM_BLK, OUT_DIM), lambda i: (i, 0)),
    out_shape=jax.ShapeDtypeStruct((TOT, OUT_DIM), jnp.float32),
)


def kernel(tokens, emb_table, W):
    tok32 = tokens.astype(jnp.int32).reshape(TOT)
    e = _sc_hash_gather(tok32, emb_table)
    out = _mm(e, W.T)
    return out.reshape(BATCH, SEQ, OUT_DIM)
